# baseline (speedup 1.0000x reference)
import jax
import jax.numpy as jnp
from jax import lax
from jax.experimental import pallas as pl
from jax.experimental.pallas import tpu as pltpu
from jax.experimental.pallas import tpu_sc as plsc

B, C, H, W = 8, 150, 224, 224
NCH = 12
NW = 32
WPB = NW // B
RPW = H // WPB
NROWS = 8
CHUNKS = RPW // NROWS
LANES = 16


def _sc_body(a_hbm, planes_hbm, out_hbm, planes_v, buf_v, out_v, sem):
    cid = lax.axis_index("c")
    sid = lax.axis_index("s")
    wid = sid * 2 + cid
    b = wid // WPB
    pr0 = (wid % WPB) * RPW

    pltpu.sync_copy(planes_hbm, planes_v)
    pvec = planes_v[b, :]
    pjs = [pvec[j] for j in range(NCH)]

    for k in range(CHUNKS):
        r0 = pr0 + k * NROWS
        cps = [
            pltpu.async_copy(
                a_hbm.at[pjs[j], pl.ds(r0, NROWS), :], buf_v.at[j], sem)
            for j in range(NCH)
        ]
        for cp in cps:
            cp.wait()

        def rbody(r, _):
            def cbody(c, _2):
                acc = buf_v[0, r, pl.ds(c * LANES, LANES)]
                for j in range(1, NCH):
                    acc = jnp.maximum(acc, buf_v[j, r, pl.ds(c * LANES, LANES)])
                out_v[r, pl.ds(c * LANES, LANES)] = acc * 5.0
                return 0
            return lax.fori_loop(0, W // LANES, cbody, 0)

        lax.fori_loop(0, NROWS, rbody, 0)
        pltpu.sync_copy(out_v, out_hbm.at[pl.ds(b * H + r0, NROWS), :])


@jax.jit
def kernel(ade_objects, ade_children_mapped):
    a3 = ade_objects.reshape(B * C, H, W)
    planes = (jnp.arange(B, dtype=jnp.int32)[:, None] * C
              + ade_children_mapped[None, :].astype(jnp.int32))
    planes = jnp.pad(planes, ((0, 0), (0, LANES - NCH)))
    run = pl.kernel(
        _sc_body,
        jax.ShapeDtypeStruct((B * H, W), jnp.float32),
        mesh=plsc.VectorSubcoreMesh(core_axis_name="c", subcore_axis_name="s"),
        scratch_types=[
            pltpu.VMEM((B, LANES), jnp.int32),
            pltpu.VMEM((NCH, NROWS, W), jnp.float32),
            pltpu.VMEM((NROWS, W), jnp.float32),
            pltpu.SemaphoreType.DMA,
        ],
    )
    out = run(a3, planes)
    return out.reshape(B, H, W)


# speedup vs baseline: 1.1300x; 1.1300x over previous
import jax
import jax.numpy as jnp
from jax import lax
from jax.experimental import pallas as pl
from jax.experimental.pallas import tpu as pltpu
from jax.experimental.pallas import tpu_sc as plsc

B, C, H, W = 8, 150, 224, 224
NCH = 12
NW = 32
WPB = NW // B
RPW = H // WPB
NROWS = 8
CHUNKS = RPW // NROWS
LANES = 16


def _tree_max(vals):
    while len(vals) > 1:
        nxt = [jnp.maximum(vals[i], vals[i + 1])
               for i in range(0, len(vals) - 1, 2)]
        if len(vals) % 2:
            nxt.append(vals[-1])
        vals = nxt
    return vals[0]


def _sc_body(a_hbm, planes_hbm, out_hbm, planes_v, buf_v, out_v,
             sem_in0, sem_in1, sem_out0, sem_out1):
    cid = lax.axis_index("c")
    sid = lax.axis_index("s")
    wid = sid * 2 + cid
    b = wid // WPB
    pr0 = (wid % WPB) * RPW

    pltpu.sync_copy(planes_hbm, planes_v)
    pvec = planes_v[b, :]
    pjs = [pvec[j] for j in range(NCH)]
    sems_in = (sem_in0, sem_in1)
    sems_out = (sem_out0, sem_out1)

    def fire(k):
        r0 = pr0 + k * NROWS
        return [
            pltpu.async_copy(
                a_hbm.at[pjs[j], pl.ds(r0, NROWS), :],
                buf_v.at[k % 2, j], sems_in[k % 2])
            for j in range(NCH)
        ]

    inflight = fire(0)
    out_cps = [None] * CHUNKS
    for k in range(CHUNKS):
        nxt = fire(k + 1) if k + 1 < CHUNKS else None
        for cp in inflight:
            cp.wait()
        if k >= 2:
            out_cps[k - 2].wait()
        p = k % 2

        def rbody(r, _):
            for c in range(W // LANES):
                sl = pl.ds(c * LANES, LANES)
                acc = _tree_max([buf_v[p, j, r, sl] for j in range(NCH)])
                out_v[p, r, sl] = acc * 5.0
            return 0

        lax.fori_loop(0, NROWS, rbody, 0)
        r0 = pr0 + k * NROWS
        out_cps[k] = pltpu.async_copy(
            out_v.at[p], out_hbm.at[pl.ds(b * H + r0, NROWS), :], sems_out[p])
        inflight = nxt
    out_cps[CHUNKS - 2].wait()
    out_cps[CHUNKS - 1].wait()


@jax.jit
def kernel(ade_objects, ade_children_mapped):
    a3 = ade_objects.reshape(B * C, H, W)
    planes = (jnp.arange(B, dtype=jnp.int32)[:, None] * C
              + ade_children_mapped[None, :].astype(jnp.int32))
    planes = jnp.pad(planes, ((0, 0), (0, LANES - NCH)))
    run = pl.kernel(
        _sc_body,
        jax.ShapeDtypeStruct((B * H, W), jnp.float32),
        mesh=plsc.VectorSubcoreMesh(core_axis_name="c", subcore_axis_name="s"),
        scratch_types=[
            pltpu.VMEM((B, LANES), jnp.int32),
            pltpu.VMEM((2, NCH, NROWS, W), jnp.float32),
            pltpu.VMEM((2, NROWS, W), jnp.float32),
            pltpu.SemaphoreType.DMA,
            pltpu.SemaphoreType.DMA,
            pltpu.SemaphoreType.DMA,
            pltpu.SemaphoreType.DMA,
        ],
    )
    out = run(a3, planes)
    return out.reshape(B, H, W)


# speedup vs baseline: 1.3022x; 1.1524x over previous
import jax
import jax.numpy as jnp
from jax import lax
from jax.experimental import pallas as pl
from jax.experimental.pallas import tpu as pltpu
from jax.experimental.pallas import tpu_sc as plsc

B, C, H, W = 8, 150, 224, 224
NCH = 12
NW = 32
WPB = NW // B
RPW = H // WPB
NROWS = 8
CHUNKS = RPW // NROWS
LANES = 16


def _tree_max(vals):
    while len(vals) > 1:
        nxt = [jnp.maximum(vals[i], vals[i + 1])
               for i in range(0, len(vals) - 1, 2)]
        if len(vals) % 2:
            nxt.append(vals[-1])
        vals = nxt
    return vals[0]


def _sc_body(a_hbm, idx_hbm, out_hbm, idx_v, buf_v, out_v,
             sem_in0, sem_in1, sem_out):
    cid = lax.axis_index("c")
    sid = lax.axis_index("s")
    wid = sid * 2 + cid
    b = wid // WPB
    pr0 = (wid % WPB) * RPW

    pltpu.sync_copy(idx_hbm, idx_v.at[pl.ds(0, NCH)])
    pvec = idx_v[...]
    base = b * C
    pjs = [pvec[j] + base for j in range(NCH)]
    sems_in = (sem_in0, sem_in1)

    def fire(k, p):
        r0 = pr0 + k * NROWS
        for j in range(NCH):
            pltpu.async_copy(
                a_hbm.at[pjs[j], pl.ds(r0, NROWS), :],
                buf_v.at[p, j], sems_in[p])

    def wait_in(p):
        pltpu.make_async_copy(
            a_hbm.at[pl.ds(0, NCH), pl.ds(0, NROWS), :],
            buf_v.at[p], sems_in[p]).wait()

    def compute_out(k, p):
        def rbody(r, _):
            for c in range(W // LANES):
                sl = pl.ds(c * LANES, LANES)
                acc = _tree_max([buf_v[p, j, r, sl] for j in range(NCH)])
                out_v[k, r, sl] = acc * 5.0
            return 0
        lax.fori_loop(0, NROWS, rbody, 0)
        pltpu.async_copy(
            out_v.at[k], out_hbm.at[pl.ds(b * H + pr0 + k * NROWS, NROWS), :],
            sem_out)

    fire(0, 0)

    def mbody(m, _):
        k0 = 2 * m
        fire(k0 + 1, 1)
        wait_in(0)
        compute_out(k0, 0)
        fire(k0 + 2, 0)
        wait_in(1)
        compute_out(k0 + 1, 1)
        return 0

    lax.fori_loop(0, (CHUNKS - 1) // 2, mbody, 0)
    wait_in(0)
    compute_out(CHUNKS - 1, 0)
    for _ in range(CHUNKS):
        pltpu.make_async_copy(
            out_v.at[0], out_hbm.at[pl.ds(0, NROWS), :], sem_out).wait()


@jax.jit
def kernel(ade_objects, ade_children_mapped):
    a3 = ade_objects.reshape(B * C, H, W)
    run = pl.kernel(
        _sc_body,
        jax.ShapeDtypeStruct((B * H, W), jnp.float32),
        mesh=plsc.VectorSubcoreMesh(core_axis_name="c", subcore_axis_name="s"),
        scratch_types=[
            pltpu.VMEM((LANES,), jnp.int32),
            pltpu.VMEM((2, NCH, NROWS, W), jnp.float32),
            pltpu.VMEM((CHUNKS, NROWS, W), jnp.float32),
            pltpu.SemaphoreType.DMA,
            pltpu.SemaphoreType.DMA,
            pltpu.SemaphoreType.DMA,
        ],
    )
    out = run(a3, ade_children_mapped.astype(jnp.int32))
    return out.reshape(B, H, W)
